# baseline (speedup 1.0000x reference)
import functools

import jax
import jax.numpy as jnp
from jax import lax
from jax.experimental import pallas as pl
from jax.experimental.pallas import tpu as pltpu
from jax.experimental.pallas import tpu_sc as plsc


def kernel(sequence_tensor, positions):
    batch_size, seq_length, width = sequence_tensor.shape
    nbatch, npos = positions.shape
    table = sequence_tensor.reshape(batch_size * seq_length, width)
    idx = positions.reshape(-1).astype(jnp.int32)
    n = nbatch * npos

    info = plsc.get_sparse_core_info()
    nc, ns, lanes = info.num_cores, info.num_subcores, info.num_lanes
    nw = nc * ns
    b_per_w = n // nw

    chunk = 40
    nchunks = b_per_w // chunk

    mesh = plsc.VectorSubcoreMesh(core_axis_name="c", subcore_axis_name="s")

    @functools.partial(
        pl.kernel,
        mesh=mesh,
        out_type=jax.ShapeDtypeStruct((n, width), jnp.float32),
        scratch_types=[
            pltpu.VMEM((b_per_w,), jnp.int32),
            pltpu.VMEM((b_per_w, width), jnp.float32),
            [pltpu.SemaphoreType.DMA] * nchunks,
            [pltpu.SemaphoreType.DMA] * nchunks,
        ],
    )
    def gather_k(table_hbm, idx_hbm, out_hbm, idx_v, rows_v, sem_g, sem_w):
        wid = lax.axis_index("s") * nc + lax.axis_index("c")
        base = wid * b_per_w
        pltpu.sync_copy(idx_hbm.at[pl.ds(base, b_per_w)], idx_v)
        offset = (base // npos) * seq_length
        for i in range(b_per_w // lanes):
            sl = pl.ds(i * lanes, lanes)
            idx_v[sl] = idx_v[sl] + offset
        gathers = []
        for k in range(nchunks):
            sl = pl.ds(k * chunk, chunk)
            gathers.append(
                pltpu.async_copy(table_hbm.at[idx_v.at[sl]], rows_v.at[sl], sem_g[k])
            )
        writes = []
        for k in range(nchunks):
            gathers[k].wait()
            sl = pl.ds(k * chunk, chunk)
            writes.append(
                pltpu.async_copy(
                    rows_v.at[sl], out_hbm.at[pl.ds(base + k * chunk, chunk)], sem_w[k]
                )
            )
        for w in writes:
            w.wait()

    return gather_k(table, idx)


# speedup vs baseline: 1.0025x; 1.0025x over previous
import functools

import jax
import jax.numpy as jnp
from jax import lax
from jax.experimental import pallas as pl
from jax.experimental.pallas import tpu as pltpu
from jax.experimental.pallas import tpu_sc as plsc


def kernel(sequence_tensor, positions):
    batch_size, seq_length, width = sequence_tensor.shape
    nbatch, npos = positions.shape
    table = sequence_tensor.reshape(batch_size * seq_length, width)
    idx = positions.reshape(-1).astype(jnp.int32)
    n = nbatch * npos

    info = plsc.get_sparse_core_info()
    nc, ns, lanes = info.num_cores, info.num_subcores, info.num_lanes
    nw = nc * ns
    b_per_w = n // nw

    chunk = 8
    nchunks = b_per_w // chunk

    mesh = plsc.VectorSubcoreMesh(core_axis_name="c", subcore_axis_name="s")

    @functools.partial(
        pl.kernel,
        mesh=mesh,
        out_type=jax.ShapeDtypeStruct((n, width), jnp.float32),
        scratch_types=[
            pltpu.VMEM((b_per_w,), jnp.int32),
            pltpu.VMEM((b_per_w, width), jnp.float32),
            [pltpu.SemaphoreType.DMA] * nchunks,
            [pltpu.SemaphoreType.DMA] * nchunks,
        ],
    )
    def gather_k(table_hbm, idx_hbm, out_hbm, idx_v, rows_v, sem_g, sem_w):
        wid = lax.axis_index("s") * nc + lax.axis_index("c")
        base = wid * b_per_w
        pltpu.sync_copy(idx_hbm.at[pl.ds(base, b_per_w)], idx_v)
        offset = (base // npos) * seq_length
        for i in range(b_per_w // lanes):
            sl = pl.ds(i * lanes, lanes)
            idx_v[sl] = idx_v[sl] + offset
        gathers = []
        for k in range(nchunks):
            sl = pl.ds(k * chunk, chunk)
            gathers.append(
                pltpu.async_copy(table_hbm.at[idx_v.at[sl]], rows_v.at[sl], sem_g[k])
            )
        writes = []
        for k in range(nchunks):
            gathers[k].wait()
            sl = pl.ds(k * chunk, chunk)
            writes.append(
                pltpu.async_copy(
                    rows_v.at[sl], out_hbm.at[pl.ds(base + k * chunk, chunk)], sem_w[k]
                )
            )
        for w in writes:
            w.wait()

    return gather_k(table, idx)


# speedup vs baseline: 1.0187x; 1.0161x over previous
import functools

import jax
import jax.numpy as jnp
from jax import lax
from jax.experimental import pallas as pl
from jax.experimental.pallas import tpu as pltpu
from jax.experimental.pallas import tpu_sc as plsc


def kernel(sequence_tensor, positions):
    batch_size, seq_length, width = sequence_tensor.shape
    nbatch, npos = positions.shape
    table = sequence_tensor.reshape(batch_size * seq_length, width)
    idx = positions.reshape(-1).astype(jnp.int32)
    n = nbatch * npos

    info = plsc.get_sparse_core_info()
    nc, ns, lanes = info.num_cores, info.num_subcores, info.num_lanes
    nw = nc * ns
    b_per_w = n // nw

    chunk = 16
    nchunks = b_per_w // chunk

    mesh = plsc.VectorSubcoreMesh(core_axis_name="c", subcore_axis_name="s")

    @functools.partial(
        pl.kernel,
        mesh=mesh,
        out_type=jax.ShapeDtypeStruct((n, width), jnp.float32),
        scratch_types=[
            pltpu.VMEM((b_per_w,), jnp.int32),
            pltpu.VMEM((b_per_w, width), jnp.float32),
            [pltpu.SemaphoreType.DMA] * nchunks,
            [pltpu.SemaphoreType.DMA] * nchunks,
        ],
    )
    def gather_k(table_hbm, idx_hbm, out_hbm, idx_v, rows_v, sem_g, sem_w):
        wid = lax.axis_index("s") * nc + lax.axis_index("c")
        base = wid * b_per_w
        pltpu.sync_copy(idx_hbm.at[pl.ds(base, b_per_w)], idx_v)
        offset = (base // npos) * seq_length
        for i in range(b_per_w // lanes):
            sl = pl.ds(i * lanes, lanes)
            idx_v[sl] = idx_v[sl] + offset
        gathers = []
        for k in range(nchunks):
            sl = pl.ds(k * chunk, chunk)
            gathers.append(
                pltpu.async_copy(table_hbm.at[idx_v.at[sl]], rows_v.at[sl], sem_g[k])
            )
        writes = []
        for k in range(nchunks):
            gathers[k].wait()
            sl = pl.ds(k * chunk, chunk)
            writes.append(
                pltpu.async_copy(
                    rows_v.at[sl], out_hbm.at[pl.ds(base + k * chunk, chunk)], sem_w[k]
                )
            )
        for w in writes:
            w.wait()

    return gather_k(table, idx)
